# baseline (speedup 1.0000x reference)
import functools

import jax
import jax.numpy as jnp
from jax import lax
from jax.experimental import pallas as pl
from jax.experimental.pallas import tpu as pltpu
from jax.experimental.pallas import tpu_sc as plsc

N_NODES_ = 10000
N_EDGES_ = 320000
FEAT_ = 128
HFEAT_ = FEAT_ // 2
NC_ = 2
NS_ = 16
CHUNK_ = 125
CHUNKS_ = 160
EDGES_PER_SUB_ = CHUNKS_ * CHUNK_
ACC_ROWS_ = N_NODES_ + 16
NBUF_ = 4
ROWS_PER_TILE_ = N_NODES_ // NS_


def _sc_agg_body(x_hbm, src_hbm, dst_hbm, zero_hbm, zcnt_hbm,
                 ones_hbm, psum_hbm, pcnt_hbm,
                 src_v, dst_v, rows0_v, rows1_v, rows2_v, rows3_v,
                 ones_v, acc_sh, cnt_sh, sem0, sem1, sem2, sem3):
    c = lax.axis_index("c")
    s = lax.axis_index("s")
    wid = c * NS_ + s
    xv = x_hbm.at[c]
    rows = (rows0_v, rows1_v, rows2_v, rows3_v)
    sems = (sem0, sem1, sem2, sem3)

    base = s * ROWS_PER_TILE_
    pltpu.sync_copy(zero_hbm, acc_sh.at[pl.ds(base, ROWS_PER_TILE_)])
    pltpu.sync_copy(zcnt_hbm, cnt_sh.at[pl.ds(base, ROWS_PER_TILE_)])

    pltpu.sync_copy(ones_hbm, ones_v)
    pltpu.sync_copy(src_hbm.at[s], src_v)
    pltpu.sync_copy(dst_hbm.at[s], dst_v)
    plsc.subcore_barrier()

    def process(m, b):
        pltpu.make_async_copy(xv.at[src_v.at[m]], rows[b], sems[b]).wait()
        pltpu.sync_copy(rows[b], acc_sh.at[dst_v.at[m]], add=True)
        @pl.when(c == b % 2)
        def _():
            pltpu.sync_copy(ones_v, cnt_sh.at[dst_v.at[m]], add=True)

    for b in range(NBUF_):
        pltpu.async_copy(xv.at[src_v.at[b]], rows[b], sems[b])

    @pl.loop(NBUF_, CHUNKS_, step=NBUF_)
    def _(j):
        for b in range(NBUF_):
            process(j - NBUF_ + b, b)
            pltpu.async_copy(xv.at[src_v.at[j + b]], rows[b], sems[b])

    for b in range(NBUF_):
        process(CHUNKS_ - NBUF_ + b, b)

    plsc.subcore_barrier()

    pltpu.sync_copy(acc_sh.at[pl.ds(base, ROWS_PER_TILE_)], psum_hbm.at[wid])
    pltpu.sync_copy(cnt_sh.at[pl.ds(base, ROWS_PER_TILE_)], pcnt_hbm.at[wid])


_sc_agg = functools.partial(
    pl.kernel,
    out_type=(
        jax.ShapeDtypeStruct((NC_ * NS_, ROWS_PER_TILE_, HFEAT_), jnp.float32),
        jax.ShapeDtypeStruct((NC_ * NS_, ROWS_PER_TILE_, 16), jnp.float32),
    ),
    mesh=plsc.VectorSubcoreMesh(core_axis_name="c", subcore_axis_name="s"),
    scratch_types=[
        pltpu.VMEM((CHUNKS_, CHUNK_), jnp.int32),
        pltpu.VMEM((CHUNKS_, CHUNK_), jnp.int32),
        pltpu.VMEM((CHUNK_, HFEAT_), jnp.float32),
        pltpu.VMEM((CHUNK_, HFEAT_), jnp.float32),
        pltpu.VMEM((CHUNK_, HFEAT_), jnp.float32),
        pltpu.VMEM((CHUNK_, HFEAT_), jnp.float32),
        pltpu.VMEM((CHUNK_, 16), jnp.float32),
        pltpu.VMEM_SHARED((ACC_ROWS_, HFEAT_), jnp.float32),
        pltpu.VMEM_SHARED((ACC_ROWS_, 16), jnp.float32),
        pltpu.SemaphoreType.DMA,
        pltpu.SemaphoreType.DMA,
        pltpu.SemaphoreType.DMA,
        pltpu.SemaphoreType.DMA,
    ],
    compiler_params=pltpu.CompilerParams(use_tc_tiling_on_sc=False),
)(_sc_agg_body)


_TC_ROWS = 2000


def _tc_prologue_body(x_ref, w_ref, nr_ref, xh_ref):
    xb = x_ref[...]
    nr = jnp.dot(xb, w_ref[...], preferred_element_type=jnp.float32,
                 precision=lax.Precision.HIGHEST)
    nr_ref[...] = jnp.maximum(nr, 0.0)
    h = _TC_ROWS // 2
    xh_ref[0] = jnp.concatenate([xb[:h, :HFEAT_], xb[h:, :HFEAT_]], axis=1)
    xh_ref[1] = jnp.concatenate([xb[:h, HFEAT_:], xb[h:, HFEAT_:]], axis=1)


def _tc_prologue(x2d, W):
    return pl.pallas_call(
        _tc_prologue_body,
        grid=(N_NODES_ // _TC_ROWS,),
        in_specs=[
            pl.BlockSpec((_TC_ROWS, FEAT_), lambda i: (i, 0)),
            pl.BlockSpec((FEAT_, FEAT_), lambda i: (0, 0)),
        ],
        out_specs=[
            pl.BlockSpec((_TC_ROWS, FEAT_), lambda i: (i, 0)),
            pl.BlockSpec((NC_, _TC_ROWS // 2, FEAT_), lambda i: (0, i, 0)),
        ],
        out_shape=[
            jax.ShapeDtypeStruct((N_NODES_, FEAT_), jnp.float32),
            jax.ShapeDtypeStruct((NC_, N_NODES_ // 2, FEAT_), jnp.float32),
        ],
    )(x2d, W)


def _tc_epilogue_body(nr_ref, w_ref, ps_ref, pc_ref, o_ref):
    ps = ps_ref[...]
    s0 = jnp.concatenate([ps[0, :, :HFEAT_], ps[0, :, HFEAT_:]], axis=0)
    s1 = jnp.concatenate([ps[1, :, :HFEAT_], ps[1, :, HFEAT_:]], axis=0)
    ssum = jnp.concatenate([s0, s1], axis=-1)
    cnt = pc_ref[...]
    agg = ssum / jnp.maximum(cnt, 1.0)
    am = jnp.dot(agg, w_ref[...], preferred_element_type=jnp.float32,
                 precision=lax.Precision.HIGHEST)
    o_ref[...] = jnp.concatenate([nr_ref[...], jnp.maximum(am, 0.0)], axis=-1)


def _tc_epilogue(nr, W, psum, pcnt):
    return pl.pallas_call(
        _tc_epilogue_body,
        grid=(N_NODES_ // _TC_ROWS,),
        in_specs=[
            pl.BlockSpec((_TC_ROWS, FEAT_), lambda i: (i, 0)),
            pl.BlockSpec((FEAT_, FEAT_), lambda i: (0, 0)),
            pl.BlockSpec((NC_, _TC_ROWS // 2, FEAT_), lambda i: (0, i, 0)),
            pl.BlockSpec((_TC_ROWS, 1), lambda i: (i, 0)),
        ],
        out_specs=pl.BlockSpec((_TC_ROWS, 2 * FEAT_), lambda i: (i, 0)),
        out_shape=jax.ShapeDtypeStruct((N_NODES_, 2 * FEAT_), jnp.float32),
    )(nr, W, psum, pcnt)


def kernel(x, edge_dst, edge_src, W):
    x2d = x.astype(jnp.float32).reshape(N_NODES_, FEAT_)

    def _perm(n):
        q = n % _TC_ROWS
        return n - q + jnp.where(q < _TC_ROWS // 2, 2 * q,
                                 2 * q - (_TC_ROWS - 1))

    src_i = edge_src.astype(jnp.int32)
    dst_i = edge_dst.astype(jnp.int32)
    src = _perm(src_i).reshape(NS_, CHUNKS_, CHUNK_)
    dst = _perm(dst_i).reshape(NS_, CHUNKS_, CHUNK_)
    iperm = _perm(jnp.arange(N_NODES_, dtype=jnp.int32))
    zero = jnp.zeros((ROWS_PER_TILE_, HFEAT_), jnp.float32)
    zcnt = jnp.zeros((ROWS_PER_TILE_, 16), jnp.float32)
    ones = jnp.ones((CHUNK_, 16), jnp.float32)
    nr, xh128 = _tc_prologue(x2d, W)
    xh = xh128.reshape(NC_, N_NODES_, HFEAT_)
    psum, pcnt = _sc_agg(xh, src, dst, zero, zcnt, ones)
    psum = psum.reshape(NC_, N_NODES_ // 2, FEAT_)
    cnt_perm = pcnt.reshape(NC_, N_NODES_, 16)[:, :, 0]
    cnt = jnp.take(cnt_perm[0] + cnt_perm[1], iperm)[:, None]
    out = _tc_epilogue(nr, W, psum, cnt)
    return out.reshape(N_NODES_, 1, 1, 2 * FEAT_)


# speedup vs baseline: 1.0144x; 1.0144x over previous
import functools

import jax
import jax.numpy as jnp
from jax import lax
from jax.experimental import pallas as pl
from jax.experimental.pallas import tpu as pltpu
from jax.experimental.pallas import tpu_sc as plsc

N_NODES_ = 10000
N_EDGES_ = 320000
FEAT_ = 128
HFEAT_ = FEAT_ // 2
NC_ = 2
NS_ = 16
CHUNK_ = 125
CHUNKS_ = 160
EDGES_PER_SUB_ = CHUNKS_ * CHUNK_
ACC_ROWS_ = N_NODES_ + 16
NBUF_ = 4
ROWS_PER_TILE_ = N_NODES_ // NS_


def _sc_agg_body(x_hbm, src_hbm, dst_hbm, zero_hbm, zcnt_hbm,
                 ones_hbm, psum_hbm, pcnt_hbm,
                 src_v, dst_v, rows0_v, rows1_v, rows2_v, rows3_v,
                 ones_v, acc_sh, cnt_sh, sem0, sem1, sem2, sem3):
    c = lax.axis_index("c")
    s = lax.axis_index("s")
    wid = c * NS_ + s
    xv = x_hbm.at[c]
    rows = (rows0_v, rows1_v, rows2_v, rows3_v)
    sems = (sem0, sem1, sem2, sem3)

    base = s * ROWS_PER_TILE_
    pltpu.sync_copy(zero_hbm, acc_sh.at[pl.ds(base, ROWS_PER_TILE_)])
    pltpu.sync_copy(zcnt_hbm, cnt_sh.at[pl.ds(base, ROWS_PER_TILE_)])

    pltpu.sync_copy(ones_hbm, ones_v)
    pltpu.sync_copy(src_hbm.at[s], src_v)
    pltpu.sync_copy(dst_hbm.at[s], dst_v)
    plsc.subcore_barrier()

    def process(m, b):
        pltpu.make_async_copy(xv.at[src_v.at[m]], rows[b], sems[b]).wait()
        pltpu.sync_copy(rows[b], acc_sh.at[dst_v.at[m]], add=True)
        @pl.when(c == b % 2)
        def _():
            pltpu.sync_copy(ones_v, cnt_sh.at[dst_v.at[m]], add=True)

    for b in range(NBUF_):
        pltpu.async_copy(xv.at[src_v.at[b]], rows[b], sems[b])

    @pl.loop(NBUF_, CHUNKS_, step=NBUF_)
    def _(j):
        for b in range(NBUF_):
            process(j - NBUF_ + b, b)
            pltpu.async_copy(xv.at[src_v.at[j + b]], rows[b], sems[b])

    for b in range(NBUF_):
        process(CHUNKS_ - NBUF_ + b, b)

    plsc.subcore_barrier()

    pltpu.sync_copy(acc_sh.at[pl.ds(base, ROWS_PER_TILE_)], psum_hbm.at[wid])
    pltpu.sync_copy(cnt_sh.at[pl.ds(base, ROWS_PER_TILE_)], pcnt_hbm.at[wid])


_sc_agg = functools.partial(
    pl.kernel,
    out_type=(
        jax.ShapeDtypeStruct((NC_ * NS_, ROWS_PER_TILE_, HFEAT_), jnp.float32),
        jax.ShapeDtypeStruct((NC_ * NS_, ROWS_PER_TILE_, 16), jnp.float32),
    ),
    mesh=plsc.VectorSubcoreMesh(core_axis_name="c", subcore_axis_name="s"),
    scratch_types=[
        pltpu.VMEM((CHUNKS_, CHUNK_), jnp.int32),
        pltpu.VMEM((CHUNKS_, CHUNK_), jnp.int32),
        pltpu.VMEM((CHUNK_, HFEAT_), jnp.float32),
        pltpu.VMEM((CHUNK_, HFEAT_), jnp.float32),
        pltpu.VMEM((CHUNK_, HFEAT_), jnp.float32),
        pltpu.VMEM((CHUNK_, HFEAT_), jnp.float32),
        pltpu.VMEM((CHUNK_, 16), jnp.float32),
        pltpu.VMEM_SHARED((ACC_ROWS_, HFEAT_), jnp.float32),
        pltpu.VMEM_SHARED((ACC_ROWS_, 16), jnp.float32),
        pltpu.SemaphoreType.DMA,
        pltpu.SemaphoreType.DMA,
        pltpu.SemaphoreType.DMA,
        pltpu.SemaphoreType.DMA,
    ],
    compiler_params=pltpu.CompilerParams(use_tc_tiling_on_sc=False),
)(_sc_agg_body)


_TC_ROWS = 2000


def _tc_prologue_body(x_ref, w_ref, nr_ref, xh_ref):
    xb = x_ref[...]
    nr = jnp.dot(xb, w_ref[...], preferred_element_type=jnp.float32,
                 precision=lax.Precision.HIGHEST)
    nr_ref[...] = jnp.maximum(nr, 0.0)
    xh_ref[0] = xb[:, :HFEAT_]
    xh_ref[1] = xb[:, HFEAT_:]


def _tc_prologue(x2d, W):
    return pl.pallas_call(
        _tc_prologue_body,
        grid=(N_NODES_ // _TC_ROWS,),
        in_specs=[
            pl.BlockSpec((_TC_ROWS, FEAT_), lambda i: (i, 0)),
            pl.BlockSpec((FEAT_, FEAT_), lambda i: (0, 0)),
        ],
        out_specs=[
            pl.BlockSpec((_TC_ROWS, FEAT_), lambda i: (i, 0)),
            pl.BlockSpec((NC_, _TC_ROWS // 2, FEAT_), lambda i: (0, i, 0)),
        ],
        out_shape=[
            jax.ShapeDtypeStruct((N_NODES_, FEAT_), jnp.float32),
            jax.ShapeDtypeStruct((NC_, N_NODES_ // 2, FEAT_), jnp.float32),
        ],
    )(x2d, W)


def _tc_epilogue_body(nr_ref, w_ref, ps_ref, pc_ref, o_ref):
    ssum = jnp.concatenate([ps_ref[0], ps_ref[1]], axis=-1)
    cnt = pc_ref[0, :, 0:1] + pc_ref[1, :, 0:1]
    agg = ssum / jnp.maximum(cnt, 1.0)
    am = jnp.dot(agg, w_ref[...], preferred_element_type=jnp.float32,
                 precision=lax.Precision.HIGHEST)
    o_ref[...] = jnp.concatenate([nr_ref[...], jnp.maximum(am, 0.0)], axis=-1)


def _tc_epilogue(nr, W, psum, pcnt):
    return pl.pallas_call(
        _tc_epilogue_body,
        grid=(N_NODES_ // _TC_ROWS,),
        in_specs=[
            pl.BlockSpec((_TC_ROWS, FEAT_), lambda i: (i, 0)),
            pl.BlockSpec((FEAT_, FEAT_), lambda i: (0, 0)),
            pl.BlockSpec((NC_, _TC_ROWS, HFEAT_), lambda i: (0, i, 0)),
            pl.BlockSpec((NC_, _TC_ROWS, 16), lambda i: (0, i, 0)),
        ],
        out_specs=pl.BlockSpec((_TC_ROWS, 2 * FEAT_), lambda i: (i, 0)),
        out_shape=jax.ShapeDtypeStruct((N_NODES_, 2 * FEAT_), jnp.float32),
    )(nr, W, psum, pcnt)


def kernel(x, edge_dst, edge_src, W):
    x2d = x.astype(jnp.float32).reshape(N_NODES_, FEAT_)
    src = edge_src.astype(jnp.int32).reshape(NS_, CHUNKS_, CHUNK_)
    dst = edge_dst.astype(jnp.int32).reshape(NS_, CHUNKS_, CHUNK_)
    zero = jnp.zeros((ROWS_PER_TILE_, HFEAT_), jnp.float32)
    zcnt = jnp.zeros((ROWS_PER_TILE_, 16), jnp.float32)
    ones = jnp.ones((CHUNK_, 16), jnp.float32)
    nr, xh = _tc_prologue(x2d, W)
    psum, pcnt = _sc_agg(xh, src, dst, zero, zcnt, ones)
    psum = psum.reshape(NC_, N_NODES_, HFEAT_)
    pcnt = pcnt.reshape(NC_, N_NODES_, 16)
    out = _tc_epilogue(nr, W, psum, pcnt)
    return out.reshape(N_NODES_, 1, 1, 2 * FEAT_)
